# baseline (speedup 1.0000x reference)
import functools

import jax
import jax.numpy as jnp
from jax import lax
from jax.experimental import pallas as pl
from jax.experimental.pallas import tpu as pltpu
from jax.experimental.pallas import tpu_sc as plsc

_BATCH = 4096
_HIST = 200
_D = 64
_DP = 128
_V = 1000000
_B = _BATCH * _HIST
_NC = 2
_NS = 16
_NW = _NC * _NS
_CHUNK = 128
_NCHUNK = _HIST
_NBUF = 4
_LEAD = 3
_NOUTER = _NCHUNK // _NBUF
_TBLK = 8192


def _relayout_body(tn_ref, out_ref):
  out_ref[:, :_D] = jnp.swapaxes(tn_ref[...], 0, 1)


_relayout = pl.pallas_call(
    _relayout_body,
    grid=(pl.cdiv(_V, _TBLK),),
    in_specs=[pl.BlockSpec((_D, _TBLK), lambda i: (0, i))],
    out_specs=pl.BlockSpec((_TBLK, _DP), lambda i: (i, 0)),
    out_shape=jax.ShapeDtypeStruct((_V, _DP), jnp.float32),
)


def _make_gather():
  mesh = plsc.VectorSubcoreMesh(core_axis_name="c", subcore_axis_name="s")

  @functools.partial(
      pl.kernel,
      mesh=mesh,
      out_type=jax.ShapeDtypeStruct((_HIST, _D // 8, _NW, 8, _DP), jnp.float32),
      compiler_params=pltpu.CompilerParams(needs_layout_passes=False),
      scratch_types=[
          pltpu.VMEM((_NCHUNK, _CHUNK), jnp.int32),
          pltpu.VMEM((_NBUF, _CHUNK, _DP), jnp.float32),
          pltpu.VMEM((_NBUF, _D // 8, 8, _DP), jnp.float32),
          pltpu.SemaphoreType.DMA((_NBUF,)),
          pltpu.SemaphoreType.DMA((_NBUF,)),
      ],
  )
  def gather_kernel(x_hbm, table_hbm, out_hbm, idx_v, rows_v, trow_v,
                    gsem, wsem):
    wid = lax.axis_index("s") * _NC + lax.axis_index("c")
    pltpu.sync_copy(x_hbm.at[wid], idx_v)
    iota16 = lax.iota(jnp.int32, 16)

    def start_gather(t, b):
      pltpu.async_copy(table_hbm.at[idx_v.at[t]], rows_v.at[b], gsem.at[b])

    def wait_gather(b):
      pltpu.make_async_copy(
          table_hbm.at[idx_v.at[0]], rows_v.at[b], gsem.at[b]).wait()

    def transpose_chunk(b):
      def tb(q, carry):
        for u in range(4):
          dtds = q * 4 + u
          dt = dtds // 8
          ds = lax.rem(dtds, 8)
          col = jnp.full((16,), dtds, jnp.int32)
          for k in range(8):
            v = plsc.load_gather(rows_v.at[b], [iota16 + (16 * k), col])
            trow_v[b, dt, ds, pl.ds(16 * k, 16)] = v
        return carry
      lax.fori_loop(0, 16, tb, 0)

    def start_write(h, b):
      pltpu.async_copy(trow_v.at[b], out_hbm.at[h, :, wid], wsem.at[b])

    def wait_write(b):
      pltpu.make_async_copy(
          trow_v.at[b], out_hbm.at[0, :, 0], wsem.at[b]).wait()

    for b in range(_LEAD):
      start_gather(b, b)

    def body(g, carry):
      for b in range(_NBUF):
        s = g * _NBUF + b
        @pl.when(g > 0)
        def _():
          wait_write(b)
        wait_gather(b)
        transpose_chunk(b)
        start_write(s, b)
        if b == 0:
          start_gather(s + _LEAD, (b + _LEAD) % _NBUF)
        else:
          @pl.when(g < _NOUTER - 1)
          def _():
            start_gather(s + _LEAD, (b + _LEAD) % _NBUF)
      return carry

    lax.fori_loop(0, _NOUTER, body, 0)

    for b in range(_NBUF):
      wait_write(b)

  return gather_kernel


_gather = _make_gather()


def kernel(x, table):
  tp = _relayout(table.T)
  xt = jnp.swapaxes(x.T.reshape(_HIST, _NW, _CHUNK), 0, 1)
  out5 = _gather(xt, tp)
  return jnp.transpose(out5, (2, 4, 0, 1, 3)).reshape(_BATCH, _HIST, _D)


# speedup vs baseline: 2.0469x; 2.0469x over previous
import functools

import jax
import jax.numpy as jnp
from jax import lax
from jax.experimental import pallas as pl
from jax.experimental.pallas import tpu as pltpu
from jax.experimental.pallas import tpu_sc as plsc

_BATCH = 4096
_HIST = 200
_D = 64
_DP = 128
_V = 1000000
_B = _BATCH * _HIST
_NC = 2
_NS = 16
_NW = _NC * _NS
_BPW = _B // _NW
_CHUNK = 128
_NCHUNK = _BPW // _CHUNK
_NBUF = 5
_LEAD = 2
_NOUTER = _NCHUNK // _NBUF
_TBLK = 16384


def _relayout_body(tn_ref, out_ref):
  out_ref[:, :_D] = jnp.swapaxes(tn_ref[...], 0, 1)


_relayout = pl.pallas_call(
    _relayout_body,
    grid=(pl.cdiv(_V, _TBLK),),
    in_specs=[pl.BlockSpec((_D, _TBLK), lambda i: (0, i))],
    out_specs=pl.BlockSpec((_TBLK, _DP), lambda i: (i, 0)),
    out_shape=jax.ShapeDtypeStruct((_V, _DP), jnp.float32),
)


def _make_gather():
  mesh = plsc.VectorSubcoreMesh(core_axis_name="c", subcore_axis_name="s")

  @functools.partial(
      pl.kernel,
      mesh=mesh,
      out_type=jax.ShapeDtypeStruct((_B, _DP), jnp.float32),
      scratch_types=[
          pltpu.VMEM((_NCHUNK, _CHUNK), jnp.int32),
          pltpu.VMEM((_NBUF, _CHUNK, _DP), jnp.float32),
          pltpu.SemaphoreType.DMA((_NBUF,)),
          pltpu.SemaphoreType.DMA((_NBUF,)),
      ],
  )
  def gather_kernel(x_hbm, table_hbm, out_hbm, idx_v, rows_v, gsem, wsem):
    wid = lax.axis_index("s") * _NC + lax.axis_index("c")
    base = wid * _BPW
    pltpu.sync_copy(x_hbm.at[wid], idx_v)

    def start_gather(t, b):
      pltpu.async_copy(table_hbm.at[idx_v.at[t]], rows_v.at[b], gsem.at[b])

    def wait_gather(b):
      pltpu.make_async_copy(
          table_hbm.at[idx_v.at[0]], rows_v.at[b], gsem.at[b]).wait()

    def start_write(t, b):
      pltpu.async_copy(
          rows_v.at[b], out_hbm.at[pl.ds(base + t * _CHUNK, _CHUNK)],
          wsem.at[b])

    def wait_write(b):
      pltpu.make_async_copy(
          rows_v.at[b], out_hbm.at[pl.ds(base, _CHUNK)], wsem.at[b]).wait()

    for b in range(_LEAD):
      start_gather(b, b)

    def body(g, carry):
      for b in range(_NBUF):
        s = g * _NBUF + b
        bg = (b + _LEAD) % _NBUF
        if b >= _NBUF - _LEAD:
          wait_write(bg)
        else:
          @pl.when(g > 0)
          def _():
            wait_write(bg)
        if b < _NBUF - _LEAD:
          start_gather(s + _LEAD, bg)
        else:
          @pl.when(g < _NOUTER - 1)
          def _():
            start_gather(s + _LEAD, bg)
        wait_gather(b)
        start_write(s, b)
      return carry

    lax.fori_loop(0, _NOUTER, body, 0)

    for t in range(_NCHUNK + _LEAD - _NBUF, _NCHUNK):
      wait_write(t % _NBUF)

  return gather_kernel


_gather = _make_gather()


def kernel(x, table):
  tp = _relayout(table.T)
  xw = x.reshape(_NW, _NCHUNK, _CHUNK)
  out = _gather(xw, tp)
  return out[:, :_D].reshape(_BATCH, _HIST, _D)
